# baseline (speedup 1.0000x reference)
import functools

import jax
import jax.numpy as jnp
from jax import lax
from jax.experimental import pallas as pl
from jax.experimental.pallas import tpu as pltpu
from jax.experimental.pallas import tpu_sc as plsc

N_ROWS = 1000001
EMB_DIM = 32
BATCH = 16384
_LANE_BLK = 128

_INFO = plsc.get_sparse_core_info()
_NC = _INFO.num_cores
_NS = _INFO.num_subcores
_NW = _NC * _NS
_B_PER_W = BATCH // _NW
_GRP = 8
_NGRP = _B_PER_W // _GRP


def _gather_body(idx_hbm, embt_hbm, outt_hbm, idx_v, cols_v, blks_v, *sems):
    wid = lax.axis_index("s") * _NC + lax.axis_index("c")
    base = wid * _B_PER_W
    pltpu.sync_copy(idx_hbm.at[pl.ds(base, _B_PER_W)], idx_v)
    lanes = lax.broadcasted_iota(jnp.int32, (16,), 0)

    def lane_scalar(v, j):
        return lax.reduce_max(jnp.where(lanes == j, v, 0), axes=(0,))

    def load_pair(h):
        return idx_v[pl.ds(h * 16, 16)]

    def fire(v, bank, lane0):
        for j in range(_GRP):
            q = lane_scalar(v >> 7, lane0 + j)
            slot = bank * _GRP + j
            pltpu.async_copy(
                embt_hbm.at[:, pl.ds(pl.multiple_of(q * _LANE_BLK, 128), _LANE_BLK)],
                blks_v.at[slot],
                sems[slot],
            )

    def drain(bank):
        for j in range(_GRP):
            slot = bank * _GRP + j
            pltpu.make_async_copy(
                embt_hbm.at[:, pl.ds(0, _LANE_BLK)], blks_v.at[slot], sems[slot]
            ).wait()

    def select(v, bank, lane0, g):
        mask = lanes < _GRP if lane0 == 0 else lanes >= _GRP
        s_vec = v & 127
        slot_vec = bank * _GRP + ((lanes - lane0) & 7)
        i_vec = g * _GRP + ((lanes - lane0) & 7)
        for c in range(EMB_DIM):
            c_vec = jnp.full((16,), c, jnp.int32)
            vals = plsc.load_gather(blks_v, [slot_vec, c_vec, s_vec])
            plsc.store_scatter(cols_v, [c_vec, i_vec], vals, mask=mask)

    def group_args(g_base, k):
        return load_pair(g_base // 2 + k // 2), 8 * (k % 2)

    def step(g_base, k, last_loop):
        g = g_base + k
        bank = k % 3
        fire_bank = (k + 2) % 3
        if not (last_loop and k >= 2):
            v2, l2 = group_args(g_base, k + 2)

            @pl.when(g + 2 < _NGRP)
            def _():
                fire(v2, fire_bank, l2)
        v, l0 = group_args(g_base, k)
        drain(bank)
        select(v, bank, l0, g)

    v0, l0 = group_args(0, 0)
    fire(v0, 0, l0)
    v1, l1 = group_args(0, 1)
    fire(v1, 1, l1)

    def loop_body(t, _):
        g_base = t * 6
        for k in range(6):
            step(g_base, k, False)
        return ()

    lax.fori_loop(0, (_NGRP - 4) // 6, loop_body, (), unroll=False)
    for k in range(4):
        step(_NGRP - 4, k, True)

    pltpu.sync_copy(cols_v, outt_hbm.at[:, pl.ds(base, _B_PER_W)])


@jax.jit
def _gather(idx, emb):
    mesh = plsc.VectorSubcoreMesh(core_axis_name="c", subcore_axis_name="s")
    run = functools.partial(
        pl.kernel,
        mesh=mesh,
        out_type=jax.ShapeDtypeStruct((EMB_DIM, BATCH), jnp.float32),
        scratch_types=[
            pltpu.VMEM((_B_PER_W,), jnp.int32),
            pltpu.VMEM((EMB_DIM, _B_PER_W), jnp.float32),
            pltpu.VMEM((3 * _GRP, EMB_DIM, _LANE_BLK), jnp.float32),
        ] + [pltpu.SemaphoreType.DMA] * (3 * _GRP),
        compiler_params=pltpu.CompilerParams(
            needs_layout_passes=False,
            disable_bounds_checks=True,
        ),
    )(_gather_body)
    out_t = run(idx, emb.T)
    return out_t.T


def kernel(idx, emb):
    return _gather(idx, emb)


# speedup vs baseline: 1.0279x; 1.0279x over previous
import functools

import jax
import jax.numpy as jnp
from jax import lax
from jax.experimental import pallas as pl
from jax.experimental.pallas import tpu as pltpu
from jax.experimental.pallas import tpu_sc as plsc

N_ROWS = 1000001
EMB_DIM = 32
BATCH = 16384
_LANE_BLK = 128

_INFO = plsc.get_sparse_core_info()
_NC = _INFO.num_cores
_NS = _INFO.num_subcores
_NW = _NC * _NS
_B_PER_W = BATCH // _NW
_GRP = 8
_NGRP = _B_PER_W // _GRP


def _gather_body(idx_hbm, embt_hbm, outt_hbm, idx_v, cols_v, blks_v, *sems):
    wid = lax.axis_index("s") * _NC + lax.axis_index("c")
    base = wid * _B_PER_W
    pltpu.sync_copy(idx_hbm.at[pl.ds(base, _B_PER_W)], idx_v)
    lanes = lax.broadcasted_iota(jnp.int32, (16,), 0)

    def lane_scalar(v, j):
        return lax.reduce_max(jnp.where(lanes == j, v, 0), axes=(0,))

    def load_pair(h):
        return idx_v[pl.ds(h * 16, 16)]

    def fire(v, bank, lane0):
        for j in range(_GRP):
            q = lane_scalar(v >> 7, lane0 + j)
            slot = bank * _GRP + j
            pltpu.async_copy(
                embt_hbm.at[:, pl.ds(pl.multiple_of(q * _LANE_BLK, 128), _LANE_BLK)],
                blks_v.at[slot],
                sems[slot],
            )

    def drain(bank):
        for j in range(_GRP):
            slot = bank * _GRP + j
            pltpu.make_async_copy(
                embt_hbm.at[:, pl.ds(0, _LANE_BLK)], blks_v.at[slot], sems[slot]
            ).wait()

    def select(v, bank, h):
        mask = lanes < _GRP if bank == 0 else lanes >= _GRP
        s_vec = v & 127
        i_vec = h * 16 + lanes
        for c in range(EMB_DIM):
            c_vec = jnp.full((16,), c, jnp.int32)
            vals = plsc.load_gather(blks_v, [lanes, c_vec, s_vec])
            plsc.store_scatter(cols_v, [c_vec, i_vec], vals, mask=mask)

    fire(load_pair(0), 0, 0)

    def loop_body(h, _):
        v = load_pair(h)
        fire(v, 1, _GRP)
        drain(0)
        select(v, 0, h)

        @pl.when(h + 1 < _NGRP // 2)
        def _():
            fire(load_pair(h + 1), 0, 0)

        drain(1)
        select(v, 1, h)
        return ()

    lax.fori_loop(0, _NGRP // 2, loop_body, (), unroll=False)

    pltpu.sync_copy(cols_v, outt_hbm.at[:, pl.ds(base, _B_PER_W)])


@jax.jit
def _gather(idx, emb):
    mesh = plsc.VectorSubcoreMesh(core_axis_name="c", subcore_axis_name="s")
    run = functools.partial(
        pl.kernel,
        mesh=mesh,
        out_type=jax.ShapeDtypeStruct((EMB_DIM, BATCH), jnp.float32),
        scratch_types=[
            pltpu.VMEM((_B_PER_W,), jnp.int32),
            pltpu.VMEM((EMB_DIM, _B_PER_W), jnp.float32),
            pltpu.VMEM((2 * _GRP, EMB_DIM, _LANE_BLK), jnp.float32),
        ] + [pltpu.SemaphoreType.DMA] * (2 * _GRP),
        compiler_params=pltpu.CompilerParams(
            needs_layout_passes=False,
            disable_bounds_checks=True,
        ),
    )(_gather_body)
    out_t = run(idx, emb.T)
    return out_t.T


def kernel(idx, emb):
    return _gather(idx, emb)
